# baseline (speedup 1.0000x reference)
import functools

import jax
import jax.numpy as jnp
from jax import lax
from jax.experimental import pallas as pl
from jax.experimental.pallas import tpu as pltpu
from jax.experimental.pallas import tpu_sc as plsc

N_FACES = 100000
N_EDGES = 150000
NNZ = 300000
D = 128

NTILES = 16
NNZP = 300032
PER_TILE = NNZP // NTILES
C = 4688
ROUNDS = 4
CHUNK = 7808
RPT = CHUNK // NTILES
TRASH = CHUNK
LROWS = 40
LTRASH = 39 * 128 + 127
BPAD = 1 << 29


def _mm_body(a_ref, w_ref, o_ref, on_ref, *, sigmoid_in):
    a = a_ref[...]
    if sigmoid_in:
        a = jax.nn.sigmoid(a)
    o = jnp.dot(a, w_ref[...], preferred_element_type=jnp.float32)
    o_ref[...] = o
    on_ref[...] = -o


def _matmul_pm(a, w, *, sigmoid_in=False, block=1000):
    n = a.shape[0]
    return pl.pallas_call(
        functools.partial(_mm_body, sigmoid_in=sigmoid_in),
        grid=(n // block,),
        in_specs=[
            pl.BlockSpec((block, D), lambda i: (i, 0)),
            pl.BlockSpec((D, D), lambda i: (0, 0)),
        ],
        out_specs=[
            pl.BlockSpec((block, D), lambda i: (i, 0)),
            pl.BlockSpec((block, D), lambda i: (i, 0)),
        ],
        out_shape=[
            jax.ShapeDtypeStruct((n, D), jnp.float32),
            jax.ShapeDtypeStruct((n, D), jnp.float32),
        ],
    )(a, w)


def _sigmoid_pallas(a, block=1000):
    n = a.shape[0]
    return pl.pallas_call(
        lambda a_ref, o_ref: o_ref.__setitem__(..., jax.nn.sigmoid(a_ref[...])),
        grid=(n // block,),
        in_specs=[pl.BlockSpec((block, D), lambda i: (i, 0))],
        out_specs=pl.BlockSpec((block, D), lambda i: (i, 0)),
        out_shape=jax.ShapeDtypeStruct((n, D), jnp.float32),
    )(a)


def _scatter_body(hpos, hneg, src_hbm, dst_hbm, val_hbm, zeros_hbm, out_hbm,
                  acc, dchunk, schunk, vchunk, ps, pd, ns, nd,
                  sstg0, sstg1, dstg0, dstg1, cb0, cb1, gb0, gb1, sem0, sem1, sem2,
                  *, NH, W):
    c = lax.axis_index("c")
    s = lax.axis_index("s")
    my_flush = s * RPT

    pltpu.sync_copy(zeros_hbm.at[pl.ds(my_flush, RPT)], acc.at[pl.ds(my_flush, RPT)])
    plsc.subcore_barrier()

    zero16 = jnp.zeros((16,), jnp.int32)
    trash16 = jnp.full((16,), TRASH, jnp.int32)
    lane = lax.iota(jnp.int32, 16)

    def init_body(r, _):
        ps[pl.ds(r * 16, 16)] = zero16
        ns[pl.ds(r * 16, 16)] = zero16
        pd[pl.ds(r * 16, 16)] = trash16
        nd[pl.ds(r * 16, 16)] = trash16
        return 0

    lax.fori_loop(0, LROWS * 8, init_body, 0)

    def window_body(w, _):
        base = jnp.minimum(w * CHUNK, NH - CHUNK) + c * NH

        def round_body(r, _r):
            def chunk_body(ci, carry):
                pcnt, ncnt = carry
                off = s * PER_TILE + r * C
                cp1 = pltpu.async_copy(dst_hbm.at[pl.ds(off, C)], dchunk, sem0)
                cp2 = pltpu.async_copy(src_hbm.at[pl.ds(off, C)], schunk, sem1)
                cp3 = pltpu.async_copy(val_hbm.at[pl.ds(off, C)], vchunk, sem2)
                cp1.wait()
                cp2.wait()
                cp3.wait()

                def grp(g, carry2):
                    pcnt, ncnt = carry2
                    d16 = dchunk[pl.ds(g * 16, 16)]
                    s16 = schunk[pl.ds(g * 16, 16)]
                    v16 = vchunk[pl.ds(g * 16, 16)]
                    loc = d16 - base
                    m_in = (d16 >= base) & (d16 < base + CHUNK)
                    m_pos = m_in & (v16 > 0.0)
                    m_neg = m_in & (v16 < 0.0)
                    keyp = jnp.where(m_pos, lane, 16 + lane)
                    keyn = jnp.where(m_neg, lane, 16 + lane)
                    _, s_p = plsc.sort_key_val(keyp, s16)
                    _, l_p = plsc.sort_key_val(keyp, loc)
                    _, s_n = plsc.sort_key_val(keyn, s16)
                    _, l_n = plsc.sort_key_val(keyn, loc)
                    ps[pl.ds(pcnt, 16)] = s_p
                    pd[pl.ds(pcnt, 16)] = jnp.minimum(l_p, TRASH)
                    ns[pl.ds(ncnt, 16)] = s_n
                    nd[pl.ds(ncnt, 16)] = jnp.minimum(l_n, TRASH)
                    pcnt = jnp.clip(
                        pcnt + plsc.all_reduce_population_count(m_pos)[0], 0, C)
                    ncnt = jnp.clip(
                        ncnt + plsc.all_reduce_population_count(m_neg)[0], 0, C)
                    return pcnt, ncnt

                return lax.fori_loop(0, C // 16, grp, (pcnt, ncnt))

            pcnt, ncnt = lax.fori_loop(0, 1, chunk_body,
                                       (jnp.int32(0), jnp.int32(0)))
            pcnt = jnp.clip(pcnt, 0, C)
            ncnt = jnp.clip(ncnt, 0, C)

            for k in range(16):
                ps[pl.ds(pcnt + k * 16, 16)] = zero16
                pd[pl.ds(pcnt + k * 16, 16)] = trash16
                ns[pl.ds(ncnt + k * 16, 16)] = zero16
                nd[pl.ds(ncnt + k * 16, 16)] = trash16

            def drain(srcmat, slist, dlist, cnt):
                nb2 = (cnt + 255) // 256

                def pair(k, _k):
                    j = k * 256
                    nsrc = srcmat.shape[0] - 1
                    for t in range(8):
                        sstg0[pl.ds(t * 16, 16)] = jnp.clip(
                            slist[pl.ds(j + t * 16, 16)], 0, nsrc)
                        sstg1[pl.ds(t * 16, 16)] = jnp.clip(
                            slist[pl.ds(j + 128 + t * 16, 16)], 0, nsrc)
                    cpa = pltpu.async_copy(srcmat.at[sstg0], gb0, sem0)
                    cpb = pltpu.async_copy(srcmat.at[sstg1], gb1, sem1)
                    for t in range(8):
                        dstg0[pl.ds(t * 16, 16)] = jnp.clip(
                            dlist[pl.ds(j + t * 16, 16)], 0, TRASH)
                        dstg1[pl.ds(t * 16, 16)] = jnp.clip(
                            dlist[pl.ds(j + 128 + t * 16, 16)], 0, TRASH)
                    cpa.wait()
                    pltpu.sync_copy(gb0, acc.at[dstg0], add=True)
                    cpb.wait()
                    pltpu.sync_copy(gb1, acc.at[dstg1], add=True)
                    return 0

                lax.fori_loop(0, nb2, pair, 0)

            DO_DRAIN = True
            if DO_DRAIN:
                drain(hpos, ps, pd, pcnt)
                drain(hneg, ns, nd, ncnt)
            return pcnt * 0 + ncnt * 0

        lax.fori_loop(0, ROUNDS, round_body, 0)

        plsc.subcore_barrier()
        pltpu.sync_copy(acc.at[pl.ds(my_flush, RPT)],
                        out_hbm.at[pl.ds(base + my_flush, RPT)])
        pltpu.sync_copy(zeros_hbm.at[pl.ds(my_flush, RPT)],
                        acc.at[pl.ds(my_flush, RPT)])
        plsc.subcore_barrier()
        return 0

    lax.fori_loop(0, W, window_body, 0)


def _sc_scatter(hpos, hneg, src_idx, dst_idx, vals, zeros, n_out):
    NH = n_out // 2
    W = -(-NH // CHUNK)
    mesh = plsc.VectorSubcoreMesh(core_axis_name="c", subcore_axis_name="s")
    f = pl.kernel(
        functools.partial(_scatter_body, NH=NH, W=W),
        out_type=jax.ShapeDtypeStruct((n_out, D), jnp.float32),
        mesh=mesh,
        compiler_params=pltpu.CompilerParams(needs_layout_passes=False),
        scratch_types=[
            pltpu.VMEM_SHARED((CHUNK + 8, D), jnp.float32),
            pltpu.VMEM((C,), jnp.int32),
            pltpu.VMEM((C,), jnp.int32),
            pltpu.VMEM((C,), jnp.float32),
            pltpu.VMEM((LROWS * 128,), jnp.int32),
            pltpu.VMEM((LROWS * 128,), jnp.int32),
            pltpu.VMEM((LROWS * 128,), jnp.int32),
            pltpu.VMEM((LROWS * 128,), jnp.int32),
            pltpu.VMEM((128,), jnp.int32),
            pltpu.VMEM((128,), jnp.int32),
            pltpu.VMEM((128,), jnp.int32),
            pltpu.VMEM((128,), jnp.int32),
            pltpu.VMEM((16,), jnp.int32),
            pltpu.VMEM((16,), jnp.int32),
            pltpu.VMEM((128, D), jnp.float32),
            pltpu.VMEM((128, D), jnp.float32),
            pltpu.SemaphoreType.DMA,
            pltpu.SemaphoreType.DMA,
            pltpu.SemaphoreType.DMA,
        ],
    )
    return f(hpos, hneg, src_idx, dst_idx, vals, zeros)


def kernel(x, rows, cols, vals, W1, W2):
    pad = NNZP - NNZ
    rows_p = jnp.concatenate([rows.astype(jnp.int32),
                              jnp.full((pad,), BPAD, jnp.int32)])
    cols_p = jnp.concatenate([cols.astype(jnp.int32),
                              jnp.zeros((pad,), jnp.int32)])
    vals_p = jnp.concatenate([vals, jnp.ones((pad,), jnp.float32)])
    zeros = jnp.zeros((CHUNK, D), jnp.float32)

    h, hn = _matmul_pm(x, W1)
    e = _sc_scatter(h, hn, cols_p, rows_p, vals_p, zeros, N_EDGES)
    h2, h2n = _matmul_pm(e, W2, sigmoid_in=True)
    o = _sc_scatter(h2, h2n, rows_p, cols_p, vals_p, zeros, N_FACES)
    return _sigmoid_pallas(o)
